# baseline (speedup 1.0000x reference)
import jax
import jax.numpy as jnp
from jax.experimental import pallas as pl

ROWS = 4096
CH = 512


def _fused_kernel(x_ref, feat_ref, w1_ref, w2_ref, pos2_ref, out_ref):
    n_cont = x_ref.shape[-1] - 3
    d6 = w1_ref.shape[1]
    nrows = x_ref.shape[0] * x_ref.shape[1]
    pos2 = pos2_ref[:]
    xall = x_ref[:].reshape(nrows, x_ref.shape[-1])
    for k in range(nrows // CH):
        lo = k * CH
        x = xall[lo:lo + CH, :]
        feat_ref[lo:lo + CH, :n_cont] = x[:, :n_cont].astype(jnp.bfloat16)
        idx = x[:, n_cont:].astype(jnp.int32)
        j = jax.lax.broadcasted_iota(jnp.int32, (CH, 128), 1)
        oh = ((j == idx[:, 1:2]) | (j == idx[:, 0:1] + 15)
              | (j == idx[:, 2:3] + 79))
        out_ref[lo:lo + CH, :d6] = jnp.dot(
            feat_ref[lo:lo + CH, :], w1_ref[:],
            preferred_element_type=jnp.float32)
        out_ref[lo:lo + CH, d6:] = jnp.dot(
            oh.astype(jnp.bfloat16), w2_ref[:],
            preferred_element_type=jnp.float32) + pos2


def kernel(x, W, b, table_dd, table_plate, table_mag, table_pos):
    B, S, F = x.shape
    n_cont = F - 3
    d6 = W.shape[1]
    d9 = table_dd.shape[1]
    d_model = d6 + 3 * d9
    N = B * S

    n_dd = table_dd.shape[0]
    n_plate = table_plate.shape[0]
    n_mag = table_mag.shape[0]

    W1e = jnp.zeros((128, d6), jnp.float32)
    W1e = W1e.at[:n_cont, :].set(W)
    W1e = W1e.at[n_cont:n_cont + S, :].set(table_pos[:S, :d6])
    W1e = W1e.at[n_cont + S, :].set(b)
    W1e = W1e.astype(jnp.bfloat16)

    W2 = jnp.zeros((128, 3 * d9), jnp.float32)
    W2 = W2.at[:n_dd, :d9].set(table_dd)
    W2 = W2.at[n_dd:n_dd + n_plate, d9:2 * d9].set(table_plate)
    W2 = W2.at[n_dd + n_plate:n_dd + n_plate + n_mag, 2 * d9:].set(table_mag)
    W2 = W2.astype(jnp.bfloat16)

    rmod = jnp.arange(ROWS, dtype=jnp.int32) % S
    feat0 = jnp.zeros((ROWS, 128), jnp.bfloat16)
    feat0 = feat0.at[:, n_cont:n_cont + S].set(
        jax.nn.one_hot(rmod, S, dtype=jnp.bfloat16))
    feat0 = feat0.at[:, n_cont + S].set(jnp.bfloat16(1))

    pos2 = jnp.tile(table_pos[:S, d6:], (CH // S, 1))

    out = pl.pallas_call(
        _fused_kernel,
        grid=(N // ROWS,),
        in_specs=[
            pl.BlockSpec((ROWS // S, S, F), lambda i: (i, 0, 0)),
            pl.BlockSpec((ROWS, 128), lambda i: (0, 0)),
            pl.BlockSpec((128, d6), lambda i: (0, 0)),
            pl.BlockSpec((128, 3 * d9), lambda i: (0, 0)),
            pl.BlockSpec((CH, 3 * d9), lambda i: (0, 0)),
        ],
        out_specs=pl.BlockSpec((ROWS, d_model), lambda i: (i, 0)),
        out_shape=jax.ShapeDtypeStruct((N, d_model), jnp.float32),
    )(x, feat0, W1e, W2, pos2)
    return out.reshape(B, S, d_model)


# speedup vs baseline: 1.1228x; 1.1228x over previous
import jax
import jax.numpy as jnp
from jax.experimental import pallas as pl

ROWS = 4096
CH = 512


def _fused_kernel(x_ref, feat_ref, w1_ref, w2_ref, pos2_ref, out_ref):
    n_cont = x_ref.shape[1] - 3
    d6 = w1_ref.shape[1]
    pos2 = pos2_ref[:]
    w1 = w1_ref[:].astype(jnp.bfloat16)
    w2 = w2_ref[:].astype(jnp.bfloat16)
    for k in range(x_ref.shape[0] // CH):
        lo = k * CH
        x = x_ref[lo:lo + CH, :]
        feat_ref[lo:lo + CH, :n_cont] = x[:, :n_cont].astype(jnp.bfloat16)
        idx = x[:, n_cont:].astype(jnp.int32)
        j = jax.lax.broadcasted_iota(jnp.int32, (CH, 128), 1)
        oh = ((j == idx[:, 1:2]) | (j == idx[:, 0:1] + 15)
              | (j == idx[:, 2:3] + 79))
        out_ref[lo:lo + CH, :d6] = jnp.dot(
            feat_ref[lo:lo + CH, :], w1,
            preferred_element_type=jnp.float32)
        out_ref[lo:lo + CH, d6:] = jnp.dot(
            oh.astype(jnp.bfloat16), w2,
            preferred_element_type=jnp.float32) + pos2


def kernel(x, W, b, table_dd, table_plate, table_mag, table_pos):
    B, S, F = x.shape
    n_cont = F - 3
    d6 = W.shape[1]
    d9 = table_dd.shape[1]
    d_model = d6 + 3 * d9
    N = B * S

    n_dd = table_dd.shape[0]
    n_plate = table_plate.shape[0]
    n_mag = table_mag.shape[0]

    W1e = jnp.zeros((128, d6), jnp.float32)
    W1e = W1e.at[:n_cont, :].set(W)
    W1e = W1e.at[n_cont:n_cont + S, :].set(table_pos[:S, :d6])
    W1e = W1e.at[n_cont + S, :].set(b)

    W2 = jnp.zeros((128, 3 * d9), jnp.float32)
    W2 = W2.at[:n_dd, :d9].set(table_dd)
    W2 = W2.at[n_dd:n_dd + n_plate, d9:2 * d9].set(table_plate)
    W2 = W2.at[n_dd + n_plate:n_dd + n_plate + n_mag, 2 * d9:].set(table_mag)

    rmod = jnp.arange(ROWS, dtype=jnp.int32) % S
    feat0 = jnp.zeros((ROWS, 128), jnp.bfloat16)
    feat0 = feat0.at[:, n_cont:n_cont + S].set(
        jax.nn.one_hot(rmod, S, dtype=jnp.bfloat16))
    feat0 = feat0.at[:, n_cont + S].set(jnp.bfloat16(1))

    pos2 = jnp.tile(table_pos[:S, d6:], (CH // S, 1))

    x2 = x.reshape(N, F)
    out = pl.pallas_call(
        _fused_kernel,
        grid=(N // ROWS,),
        in_specs=[
            pl.BlockSpec((ROWS, F), lambda i: (i, 0)),
            pl.BlockSpec((ROWS, 128), lambda i: (0, 0)),
            pl.BlockSpec((128, d6), lambda i: (0, 0)),
            pl.BlockSpec((128, 3 * d9), lambda i: (0, 0)),
            pl.BlockSpec((CH, 3 * d9), lambda i: (0, 0)),
        ],
        out_specs=pl.BlockSpec((ROWS, d_model), lambda i: (i, 0)),
        out_shape=jax.ShapeDtypeStruct((N, d_model), jnp.float32),
    )(x2, feat0, W1e, W2, pos2)
    return out.reshape(B, S, d_model)


# speedup vs baseline: 1.1235x; 1.0006x over previous
import jax
import jax.numpy as jnp
from jax.experimental import pallas as pl
from jax.experimental.pallas import tpu as pltpu

ROWS = 4096
CH = 512


def _fused_kernel(x_ref, w_ref, b_ref, dd_ref, plate_ref, mag_ref, pos_ref,
                  out_ref, feat_ref, w1_ref, w2_ref, pos2_ref):
    n_cont = x_ref.shape[1] - 3
    d6 = w_ref.shape[1]
    s = pos_ref.shape[0]
    n_dd = dd_ref.shape[0]
    n_plate = plate_ref.shape[0]
    n_mag = mag_ref.shape[0]

    @pl.when(pl.program_id(0) == 0)
    def _assemble():
        w1_ref[:] = jnp.zeros_like(w1_ref)
        w1_ref[:n_cont, :] = w_ref[:].astype(jnp.bfloat16)
        w1_ref[n_cont:n_cont + s, :] = pos_ref[:, :d6].astype(jnp.bfloat16)
        w1_ref[n_cont + s:n_cont + s + 1, :] = b_ref[:].astype(jnp.bfloat16)
        d9 = dd_ref.shape[1]
        w2_ref[:] = jnp.zeros_like(w2_ref)
        w2_ref[:n_dd, :d9] = dd_ref[:].astype(jnp.bfloat16)
        w2_ref[n_dd:n_dd + n_plate, d9:2 * d9] = (
            plate_ref[:].astype(jnp.bfloat16))
        w2_ref[n_dd + n_plate:n_dd + n_plate + n_mag, 2 * d9:] = (
            mag_ref[:].astype(jnp.bfloat16))
        i0 = jax.lax.broadcasted_iota(jnp.int32, feat_ref.shape, 0)
        j0 = jax.lax.broadcasted_iota(jnp.int32, feat_ref.shape, 1)
        tail = (j0 - n_cont == (i0 & (s - 1))) | (j0 == n_cont + s)
        feat_ref[:] = tail.astype(jnp.bfloat16)
        for t in range(CH // s):
            pos2_ref[t * s:(t + 1) * s, :] = pos_ref[:, d6:]

    pos2 = pos2_ref[:]
    w1 = w1_ref[:]
    w2 = w2_ref[:]
    for k in range(x_ref.shape[0] // CH):
        lo = k * CH
        x = x_ref[lo:lo + CH, :]
        feat_ref[lo:lo + CH, :n_cont] = x[:, :n_cont].astype(jnp.bfloat16)
        idx = x[:, n_cont:].astype(jnp.int32)
        j = jax.lax.broadcasted_iota(jnp.int32, (CH, 128), 1)
        oh = ((j == idx[:, 1:2]) | (j == idx[:, 0:1] + n_dd)
              | (j == idx[:, 2:3] + n_dd + n_plate))
        out_ref[lo:lo + CH, :d6] = jnp.dot(
            feat_ref[lo:lo + CH, :], w1,
            preferred_element_type=jnp.float32)
        out_ref[lo:lo + CH, d6:] = jnp.dot(
            oh.astype(jnp.bfloat16), w2,
            preferred_element_type=jnp.float32) + pos2


def kernel(x, W, b, table_dd, table_plate, table_mag, table_pos):
    B, S, F = x.shape
    d6 = W.shape[1]
    d9 = table_dd.shape[1]
    d_model = d6 + 3 * d9
    N = B * S

    x2 = x.reshape(N, F)
    b2 = b.reshape(1, d6)
    out = pl.pallas_call(
        _fused_kernel,
        grid=(N // ROWS,),
        in_specs=[
            pl.BlockSpec((ROWS, F), lambda i: (i, 0)),
            pl.BlockSpec(W.shape, lambda i: (0, 0)),
            pl.BlockSpec((1, d6), lambda i: (0, 0)),
            pl.BlockSpec(table_dd.shape, lambda i: (0, 0)),
            pl.BlockSpec(table_plate.shape, lambda i: (0, 0)),
            pl.BlockSpec(table_mag.shape, lambda i: (0, 0)),
            pl.BlockSpec((S, d_model), lambda i: (0, 0)),
        ],
        out_specs=pl.BlockSpec((ROWS, d_model), lambda i: (i, 0)),
        out_shape=jax.ShapeDtypeStruct((N, d_model), jnp.float32),
        scratch_shapes=[
            pltpu.VMEM((ROWS, 128), jnp.bfloat16),
            pltpu.VMEM((128, d6), jnp.bfloat16),
            pltpu.VMEM((128, 3 * d9), jnp.bfloat16),
            pltpu.VMEM((CH, 3 * d9), jnp.float32),
        ],
    )(x2, W, b2, table_dd, table_plate, table_mag, table_pos)
    return out.reshape(B, S, d_model)
